# baseline (speedup 1.0000x reference)
import functools

import jax
import jax.numpy as jnp
from jax import lax
from jax.experimental import pallas as pl
from jax.experimental.pallas import tpu as pltpu
from jax.experimental.pallas import tpu_sc as plsc

N_NODES = 10000
D = 128
B, S = 4096, 20
BS = B * S

NC, NS = 2, 16
NW = NC * NS
BPW = BS // NW
CH = 128
NCH = BPW // CH

TC_ROWS = 2560
TC_GRID = BS // TC_ROWS


@functools.lru_cache(maxsize=1)
def _sc_rows_call():
    @functools.partial(
        pl.kernel,
        mesh=plsc.VectorSubcoreMesh(core_axis_name="c", subcore_axis_name="s"),
        out_type=[
            jax.ShapeDtypeStruct((BS, D), jnp.float32),
            jax.ShapeDtypeStruct((BS, D), jnp.float32),
        ],
        scratch_types=[
            pltpu.VMEM((NCH, CH), jnp.int32),
            pltpu.VMEM((CH, D), jnp.float32),
            pltpu.VMEM((CH, D), jnp.float32),
            pltpu.SemaphoreType.DMA,
            pltpu.SemaphoreType.DMA,
        ],
    )
    def _sc_rows(sidx_hbm, de_hbm, pre_hbm, de_out, pre_out,
                 sidx_v, de_b, pre_b, sem_de, sem_pre):
        wid = lax.axis_index("s") * NC + lax.axis_index("c")
        base = wid * BPW
        pltpu.sync_copy(sidx_hbm.at[wid], sidx_v)
        for j in range(NCH):
            ce = pltpu.async_copy(de_hbm.at[sidx_v.at[j]], de_b, sem_de)
            cp = pltpu.async_copy(pre_hbm.at[sidx_v.at[j]], pre_b, sem_pre)
            ce.wait()
            pltpu.sync_copy(de_b, de_out.at[pl.ds(base + j * CH, CH)])
            cp.wait()
            pltpu.sync_copy(pre_b, pre_out.at[pl.ds(base + j * CH, CH)])

    return _sc_rows


@functools.lru_cache(maxsize=1)
def _sc_dist_call():
    @functools.partial(
        pl.kernel,
        mesh=plsc.VectorSubcoreMesh(core_axis_name="c", subcore_axis_name="s"),
        out_type=jax.ShapeDtypeStruct((BS,), jnp.float32),
        scratch_types=[
            pltpu.VMEM((NCH, CH), jnp.int32),
            pltpu.VMEM((CH,), jnp.float32),
            pltpu.SemaphoreType.DMA,
        ],
    )
    def _sc_dist(fidx_hbm, sp_hbm, dist_out, fidx_v, dist_b, sem_d):
        wid = lax.axis_index("s") * NC + lax.axis_index("c")
        base = wid * BPW
        pltpu.sync_copy(fidx_hbm.at[wid], fidx_v)
        for j in range(NCH):
            cd = pltpu.async_copy(sp_hbm.at[fidx_v.at[j]], dist_b, sem_d)
            cd.wait()
            pltpu.sync_copy(dist_b, dist_out.at[pl.ds(base + j * CH, CH)])

    return _sc_dist


def _tc_body(dist_ref, raw_ref, de_ref, pre_ref,
             sw1_ref, sb1_ref, sw2_ref, sb2_ref,
             cw1_ref, cb1_ref, cw2_ref, cb2_ref, out_ref):
    f32 = jnp.float32
    bf16 = jnp.bfloat16
    cw1 = cw1_ref[...].astype(bf16)
    w1r = cw1[0:D]
    w1s = cw1[D:2 * D]
    w1d = cw1[2 * D:3 * D]
    w1p = cw1[3 * D:4 * D]
    u = jnp.maximum(dist_ref[...] * sw1_ref[...] + sb1_ref[...],
                    0.0).astype(bf16)
    m = jnp.dot(sw2_ref[...].astype(bf16), w1s, preferred_element_type=f32)
    h = jnp.dot(raw_ref[...].astype(bf16), w1r, preferred_element_type=f32)
    h = h + jnp.dot(de_ref[...].astype(bf16), w1d, preferred_element_type=f32)
    h = h + jnp.dot(pre_ref[...].astype(bf16), w1p, preferred_element_type=f32)
    h = h + jnp.dot(u, m.astype(bf16), preferred_element_type=f32)
    h = h + cb1_ref[...] + jnp.dot(sb2_ref[...], cw1_ref[...][D:2 * D],
                                   preferred_element_type=f32)
    out_ref[...] = (jnp.dot(jnp.maximum(h, 0.0).astype(bf16),
                            cw2_ref[...].astype(bf16),
                            preferred_element_type=f32) + cb2_ref[...])


def _tc_specs():
    full = lambda shape: pl.BlockSpec(shape, lambda i: (0,) * len(shape))
    row = lambda shape: pl.BlockSpec(shape, lambda i: (i,) + (0,) * (len(shape) - 1))
    in_specs = [
        row((TC_ROWS, 1)),
        row((TC_ROWS, D)),
        row((TC_ROWS, D)),
        row((TC_ROWS, D)),
        full((1, D // 2)),
        full((1, D // 2)),
        full((D // 2, D)),
        full((1, D)),
        full((4 * D, 2 * D)),
        full((1, 2 * D)),
        full((2 * D, D)),
        full((1, D)),
    ]
    return in_specs, row((TC_ROWS, D))


def _tc_call(interpret=False):
    in_specs, out_spec = _tc_specs()
    return pl.pallas_call(
        _tc_body,
        grid=(TC_GRID,),
        in_specs=in_specs,
        out_specs=out_spec,
        out_shape=jax.ShapeDtypeStruct((BS, D), jnp.float32),
        compiler_params=pltpu.CompilerParams(
            dimension_semantics=("arbitrary",)),
        interpret=interpret,
    )


def kernel(node_i, sample_nodes, raw_embed, shortest_paths, de_cache, pre_cache,
           spe_w1, spe_b1, spe_w2, spe_b2, comb_w1, comb_b1, comb_w2, comb_b2):
    samp_t = sample_nodes.astype(jnp.int32).T
    flat_t = node_i.astype(jnp.int32)[None, :] * N_NODES + samp_t
    de_g, pre_g = _sc_rows_call()(
        samp_t.reshape(NW, NCH, CH), de_cache, pre_cache)
    dist = _sc_dist_call()(
        flat_t.reshape(NW, NCH, CH),
        shortest_paths.reshape(N_NODES * N_NODES))
    raw2 = raw_embed.transpose(1, 0, 2).reshape(BS, D)
    out = _tc_call()(
        dist.reshape(BS, 1), raw2, de_g, pre_g,
        spe_w1, spe_b1.reshape(1, D // 2), spe_w2, spe_b2.reshape(1, D),
        comb_w1, comb_b1.reshape(1, 2 * D), comb_w2, comb_b2.reshape(1, D))
    return out.reshape(S, B, D).transpose(1, 0, 2)


# speedup vs baseline: 1.0070x; 1.0070x over previous
import functools

import jax
import jax.numpy as jnp
from jax import lax
from jax.experimental import pallas as pl
from jax.experimental.pallas import tpu as pltpu
from jax.experimental.pallas import tpu_sc as plsc

N_NODES = 10000
D = 128
B, S = 4096, 20
BS = B * S

NC, NS = 2, 16
NW = NC * NS
BPW = BS // NW
CH = 128
NCH = BPW // CH

TC_ROWS = 2560
TC_GRID = BS // TC_ROWS


@functools.lru_cache(maxsize=1)
def _sc_gather_call():
    @functools.partial(
        pl.kernel,
        mesh=plsc.VectorSubcoreMesh(core_axis_name="c", subcore_axis_name="s"),
        out_type=[
            jax.ShapeDtypeStruct((BS,), jnp.float32),
            jax.ShapeDtypeStruct((BS, D), jnp.float32),
            jax.ShapeDtypeStruct((BS, D), jnp.float32),
        ],
        scratch_types=[
            pltpu.VMEM((NCH, CH), jnp.int32),
            pltpu.VMEM((NCH, CH), jnp.int32),
            pltpu.VMEM((CH,), jnp.float32),
            pltpu.VMEM((CH, D), jnp.float32),
            pltpu.VMEM((CH, D), jnp.float32),
            pltpu.SemaphoreType.DMA,
            pltpu.SemaphoreType.DMA,
            pltpu.SemaphoreType.DMA,
        ],
    )
    def _sc_gather(fidx_hbm, sidx_hbm, sp_hbm, de_hbm, pre_hbm,
                   dist_out, de_out, pre_out,
                   fidx_v, sidx_v, dist_b, de_b, pre_b, sem_d, sem_de, sem_pre):
        wid = lax.axis_index("s") * NC + lax.axis_index("c")
        base = wid * BPW
        pltpu.sync_copy(fidx_hbm.at[wid], fidx_v)
        pltpu.sync_copy(sidx_hbm.at[wid], sidx_v)
        for j in range(NCH):
            cd = pltpu.async_copy(sp_hbm.at[fidx_v.at[j]], dist_b, sem_d)
            ce = pltpu.async_copy(de_hbm.at[sidx_v.at[j]], de_b, sem_de)
            cp = pltpu.async_copy(pre_hbm.at[sidx_v.at[j]], pre_b, sem_pre)
            cd.wait()
            pltpu.sync_copy(dist_b, dist_out.at[pl.ds(base + j * CH, CH)])
            ce.wait()
            pltpu.sync_copy(de_b, de_out.at[pl.ds(base + j * CH, CH)])
            cp.wait()
            pltpu.sync_copy(pre_b, pre_out.at[pl.ds(base + j * CH, CH)])

    return _sc_gather


def _tc_body(dist_ref, raw_ref, de_ref, pre_ref,
             sw1_ref, sb1_ref, sw2_ref, sb2_ref,
             cw1_ref, cb1_ref, cw2_ref, cb2_ref, out_ref):
    f32 = jnp.float32
    cw1 = cw1_ref[...]
    w1r = cw1[0:D]
    w1s = cw1[D:2 * D]
    w1d = cw1[2 * D:3 * D]
    w1p = cw1[3 * D:4 * D]
    u = jnp.maximum(dist_ref[...] * sw1_ref[...] + sb1_ref[...], 0.0)
    m = jnp.dot(sw2_ref[...], w1s, preferred_element_type=f32)
    h = jnp.dot(raw_ref[...], w1r, preferred_element_type=f32)
    h = h + jnp.dot(de_ref[...], w1d, preferred_element_type=f32)
    h = h + jnp.dot(pre_ref[...], w1p, preferred_element_type=f32)
    h = h + jnp.dot(u, m, preferred_element_type=f32)
    h = h + cb1_ref[...] + jnp.dot(sb2_ref[...], w1s, preferred_element_type=f32)
    out_ref[...] = (jnp.dot(jnp.maximum(h, 0.0), cw2_ref[...],
                            preferred_element_type=f32) + cb2_ref[...])


def _tc_specs():
    full = lambda shape: pl.BlockSpec(shape, lambda i: (0,) * len(shape))
    row = lambda shape: pl.BlockSpec(shape, lambda i: (i,) + (0,) * (len(shape) - 1))
    in_specs = [
        row((TC_ROWS, 1)),
        row((TC_ROWS, D)),
        row((TC_ROWS, D)),
        row((TC_ROWS, D)),
        full((1, D // 2)),
        full((1, D // 2)),
        full((D // 2, D)),
        full((1, D)),
        full((4 * D, 2 * D)),
        full((1, 2 * D)),
        full((2 * D, D)),
        full((1, D)),
    ]
    return in_specs, row((TC_ROWS, D))


def _tc_call(interpret=False):
    in_specs, out_spec = _tc_specs()
    return pl.pallas_call(
        _tc_body,
        grid=(TC_GRID,),
        in_specs=in_specs,
        out_specs=out_spec,
        out_shape=jax.ShapeDtypeStruct((BS, D), jnp.float32),
        compiler_params=pltpu.CompilerParams(
            dimension_semantics=("arbitrary",)),
        interpret=interpret,
    )


def kernel(node_i, sample_nodes, raw_embed, shortest_paths, de_cache, pre_cache,
           spe_w1, spe_b1, spe_w2, spe_b2, comb_w1, comb_b1, comb_w2, comb_b2):
    samp_t = sample_nodes.astype(jnp.int32).T
    flat_t = node_i.astype(jnp.int32)[None, :] * N_NODES + samp_t
    dist, de_g, pre_g = _sc_gather_call()(
        flat_t.reshape(NW, NCH, CH),
        samp_t.reshape(NW, NCH, CH),
        shortest_paths.reshape(N_NODES * N_NODES),
        de_cache, pre_cache)
    raw2 = raw_embed.transpose(1, 0, 2).reshape(BS, D)
    out = _tc_call()(
        dist.reshape(BS, 1), raw2, de_g, pre_g,
        spe_w1, spe_b1.reshape(1, D // 2), spe_w2, spe_b2.reshape(1, D),
        comb_w1, comb_b1.reshape(1, 2 * D), comb_w2, comb_b2.reshape(1, D))
    return out.reshape(S, B, D).transpose(1, 0, 2)


# speedup vs baseline: 1.0261x; 1.0190x over previous
import functools

import jax
import jax.numpy as jnp
from jax import lax
from jax.experimental import pallas as pl
from jax.experimental.pallas import tpu as pltpu
from jax.experimental.pallas import tpu_sc as plsc

N_NODES = 10000
D = 128
B, S = 4096, 20
BS = B * S

NC, NS = 2, 16
NW = NC * NS
BPW = BS // NW
CH = 128
NCH = BPW // CH

TC_ROWS = 2560
TC_GRID = BS // TC_ROWS


@functools.lru_cache(maxsize=1)
def _sc_gather_call():
    @functools.partial(
        pl.kernel,
        mesh=plsc.VectorSubcoreMesh(core_axis_name="c", subcore_axis_name="s"),
        out_type=[
            jax.ShapeDtypeStruct((BS,), jnp.float32),
            jax.ShapeDtypeStruct((BS, D), jnp.float32),
            jax.ShapeDtypeStruct((BS, D), jnp.float32),
        ],
        scratch_types=[
            pltpu.VMEM((NCH, CH), jnp.int32),
            pltpu.VMEM((NCH, CH), jnp.int32),
            pltpu.VMEM((2, CH), jnp.float32),
            pltpu.VMEM((2, CH, D), jnp.float32),
            pltpu.VMEM((2, CH, D), jnp.float32),
            pltpu.SemaphoreType.DMA,
            pltpu.SemaphoreType.DMA,
            pltpu.SemaphoreType.DMA,
            pltpu.SemaphoreType.DMA,
            pltpu.SemaphoreType.DMA,
            pltpu.SemaphoreType.DMA,
        ],
    )
    def _sc_gather(fidx_hbm, sidx_hbm, sp_hbm, de_hbm, pre_hbm,
                   dist_out, de_out, pre_out,
                   fidx_v, sidx_v, dist_b, de_b, pre_b, *sems):
        wid = lax.axis_index("s") * NC + lax.axis_index("c")
        base = wid * BPW
        pltpu.sync_copy(fidx_hbm.at[wid], fidx_v)
        pltpu.sync_copy(sidx_hbm.at[wid], sidx_v)

        def fire(j, p):
            return (
                pltpu.async_copy(sp_hbm.at[fidx_v.at[j]], dist_b.at[p],
                                 sems[3 * p]),
                pltpu.async_copy(de_hbm.at[sidx_v.at[j]], de_b.at[p],
                                 sems[3 * p + 1]),
                pltpu.async_copy(pre_hbm.at[sidx_v.at[j]], pre_b.at[p],
                                 sems[3 * p + 2]),
            )

        inflight = fire(0, 0)
        for j in range(NCH):
            p = j % 2
            cur = inflight
            if j + 1 < NCH:
                inflight = fire(j + 1, 1 - p)
            for c in cur:
                c.wait()
            pltpu.sync_copy(dist_b.at[p], dist_out.at[pl.ds(base + j * CH, CH)])
            pltpu.sync_copy(de_b.at[p], de_out.at[pl.ds(base + j * CH, CH)])
            pltpu.sync_copy(pre_b.at[p], pre_out.at[pl.ds(base + j * CH, CH)])

    return _sc_gather


def _tc_body(dist_ref, raw_ref, de_ref, pre_ref,
             sw1_ref, sb1_ref, sw2_ref, sb2_ref,
             cw1_ref, cb1_ref, cw2_ref, cb2_ref, out_ref):
    f32 = jnp.float32
    cw1 = cw1_ref[...]
    w1r = cw1[0:D]
    w1s = cw1[D:2 * D]
    w1d = cw1[2 * D:3 * D]
    w1p = cw1[3 * D:4 * D]
    u = jnp.maximum(dist_ref[...] * sw1_ref[...] + sb1_ref[...], 0.0)
    m = jnp.dot(sw2_ref[...], w1s, preferred_element_type=f32)
    h = jnp.dot(raw_ref[...], w1r, preferred_element_type=f32)
    h = h + jnp.dot(de_ref[...], w1d, preferred_element_type=f32)
    h = h + jnp.dot(pre_ref[...], w1p, preferred_element_type=f32)
    h = h + jnp.dot(u, m, preferred_element_type=f32)
    h = h + cb1_ref[...] + jnp.dot(sb2_ref[...], w1s, preferred_element_type=f32)
    out_ref[...] = (jnp.dot(jnp.maximum(h, 0.0), cw2_ref[...],
                            preferred_element_type=f32) + cb2_ref[...])


def _tc_specs():
    full = lambda shape: pl.BlockSpec(shape, lambda i: (0,) * len(shape))
    row = lambda shape: pl.BlockSpec(shape, lambda i: (i,) + (0,) * (len(shape) - 1))
    in_specs = [
        row((TC_ROWS, 1)),
        row((TC_ROWS, D)),
        row((TC_ROWS, D)),
        row((TC_ROWS, D)),
        full((1, D // 2)),
        full((1, D // 2)),
        full((D // 2, D)),
        full((1, D)),
        full((4 * D, 2 * D)),
        full((1, 2 * D)),
        full((2 * D, D)),
        full((1, D)),
    ]
    return in_specs, row((TC_ROWS, D))


def _tc_call(interpret=False):
    in_specs, out_spec = _tc_specs()
    return pl.pallas_call(
        _tc_body,
        grid=(TC_GRID,),
        in_specs=in_specs,
        out_specs=out_spec,
        out_shape=jax.ShapeDtypeStruct((BS, D), jnp.float32),
        compiler_params=pltpu.CompilerParams(
            dimension_semantics=("arbitrary",)),
        interpret=interpret,
    )


def kernel(node_i, sample_nodes, raw_embed, shortest_paths, de_cache, pre_cache,
           spe_w1, spe_b1, spe_w2, spe_b2, comb_w1, comb_b1, comb_w2, comb_b2):
    samp_t = sample_nodes.astype(jnp.int32).T
    flat_t = node_i.astype(jnp.int32)[None, :] * N_NODES + samp_t
    dist, de_g, pre_g = _sc_gather_call()(
        flat_t.reshape(NW, NCH, CH),
        samp_t.reshape(NW, NCH, CH),
        shortest_paths.reshape(N_NODES * N_NODES),
        de_cache, pre_cache)
    raw2 = raw_embed.transpose(1, 0, 2).reshape(BS, D)
    out = _tc_call()(
        dist.reshape(BS, 1), raw2, de_g, pre_g,
        spe_w1, spe_b1.reshape(1, D // 2), spe_w2, spe_b2.reshape(1, D),
        comb_w1, comb_b1.reshape(1, 2 * D), comb_w2, comb_b2.reshape(1, D))
    return out.reshape(S, B, D).transpose(1, 0, 2)
